# baseline (speedup 1.0000x reference)
import functools
import math

import jax
import jax.numpy as jnp
from jax import lax
from jax.experimental import pallas as pl
from jax.experimental.pallas import tpu as pltpu
from jax.experimental.pallas import tpu_sc as plsc

D_MODEL = 64
SCALE = math.sqrt(D_MODEL)

NUM_CORES = 2
NUM_SUBCORES = 16
NW = NUM_CORES * NUM_SUBCORES
LANES = 16
CHUNK = 512


@functools.cache
def _build(B: int):
    assert B % (NW * CHUNK) == 0
    b_per_w = B // NW
    n_chunks = b_per_w // CHUNK
    assert n_chunks % 2 == 0
    mesh = plsc.VectorSubcoreMesh(
        core_axis_name="c", subcore_axis_name="s",
        num_cores=NUM_CORES, num_subcores=NUM_SUBCORES)

    @functools.partial(
        pl.kernel,
        out_type=jax.ShapeDtypeStruct((B, D_MODEL), jnp.float32),
        mesh=mesh,
        scratch_types=[
            pltpu.VMEM((b_per_w,), jnp.int32),
            pltpu.VMEM((CHUNK, D_MODEL), jnp.float32),
            pltpu.VMEM((CHUNK, D_MODEL), jnp.float32),
            pltpu.SemaphoreType.DMA,
            pltpu.SemaphoreType.DMA,
        ],
        compiler_params=pltpu.CompilerParams(use_tc_tiling_on_sc=False),
    )
    def emb(x_hbm, w_hbm, out_hbm, idx_v, rows0, rows1, gsem0, gsem1):
        rows = (rows0, rows1)
        gsem = (gsem0, gsem1)
        wid = lax.axis_index("s") * NUM_CORES + lax.axis_index("c")
        base = wid * b_per_w

        pltpu.sync_copy(x_hbm.at[pl.ds(base, b_per_w)], idx_v)

        pltpu.async_copy(w_hbm.at[idx_v.at[pl.ds(0, CHUNK)]], rows0, gsem0)

        @pl.loop(0, n_chunks, step=2)
        def _chunks(g0):
            for b in range(2):
                g = g0 + b
                pltpu.make_async_copy(
                    w_hbm.at[idx_v.at[pl.ds(0, CHUNK)]], rows[b], gsem[b]
                ).wait()

                @pl.when(g + 1 < n_chunks)
                def _():
                    pltpu.async_copy(
                        w_hbm.at[idx_v.at[pl.ds((g + 1) * CHUNK, CHUNK)]],
                        rows[1 - b], gsem[1 - b])

                @pl.loop(0, CHUNK)
                def _scale(i):
                    for j in range(D_MODEL // LANES):
                        sl = pl.ds(j * LANES, LANES)
                        rows[b][i, sl] = rows[b][i, sl] * SCALE

                pltpu.sync_copy(rows[b],
                                out_hbm.at[pl.ds(base + g * CHUNK, CHUNK)])

    return emb


def kernel(x, weight):
    batch, seq = x.shape
    flat = x.reshape(-1).astype(jnp.int32)
    out = _build(batch * seq)(flat, weight)
    return out.reshape(batch, seq, D_MODEL)


# speedup vs baseline: 1.0247x; 1.0247x over previous
import functools
import math

import jax
import jax.numpy as jnp
from jax import lax
from jax.experimental import pallas as pl
from jax.experimental.pallas import tpu as pltpu
from jax.experimental.pallas import tpu_sc as plsc

D_MODEL = 64
SCALE = math.sqrt(D_MODEL)

NUM_CORES = 2
NUM_SUBCORES = 16
NW = NUM_CORES * NUM_SUBCORES
LANES = 16
CHUNK = 256
NBUF = 4


@functools.cache
def _build(B: int):
    assert B % (NW * CHUNK) == 0
    b_per_w = B // NW
    n_chunks = b_per_w // CHUNK
    assert n_chunks % NBUF == 0 and n_chunks >= 2 * NBUF
    mesh = plsc.VectorSubcoreMesh(
        core_axis_name="c", subcore_axis_name="s",
        num_cores=NUM_CORES, num_subcores=NUM_SUBCORES)

    @functools.partial(
        pl.kernel,
        out_type=jax.ShapeDtypeStruct((B, D_MODEL), jnp.float32),
        mesh=mesh,
        scratch_types=[
            pltpu.VMEM((b_per_w,), jnp.int32),
            [pltpu.VMEM((CHUNK, D_MODEL), jnp.float32)] * NBUF,
            [pltpu.SemaphoreType.DMA] * NBUF,
            [pltpu.SemaphoreType.DMA] * NBUF,
        ],
        compiler_params=pltpu.CompilerParams(use_tc_tiling_on_sc=False),
    )
    def emb(x_hbm, w_hbm, out_hbm, idx_v, rows, gsem, ssem):
        wid = lax.axis_index("s") * NUM_CORES + lax.axis_index("c")
        base = wid * b_per_w

        def gather(g, b):
            pltpu.async_copy(
                w_hbm.at[idx_v.at[pl.ds(g * CHUNK, CHUNK)]], rows[b], gsem[b])

        def wait_gather(b):
            pltpu.make_async_copy(
                w_hbm.at[idx_v.at[pl.ds(0, CHUNK)]], rows[b], gsem[b]).wait()

        def scatter(g, b):
            pltpu.async_copy(
                rows[b], out_hbm.at[pl.ds(base + g * CHUNK, CHUNK)], ssem[b])

        def wait_scatter(b):
            pltpu.make_async_copy(
                rows[b], out_hbm.at[pl.ds(base, CHUNK)], ssem[b]).wait()

        pltpu.sync_copy(x_hbm.at[pl.ds(base, b_per_w)], idx_v)

        for b in range(2):
            gather(b, b)

        @pl.loop(0, n_chunks, step=NBUF)
        def _chunks(g0):
            for b in range(NBUF):
                g = g0 + b
                pf = (b + 2) % NBUF
                @pl.when(g + 2 < n_chunks)
                def _():
                    @pl.when(g >= 2)
                    def _():
                        wait_scatter(pf)
                    gather(g + 2, pf)

                wait_gather(b)

                @plsc.parallel_loop(0, CHUNK, unroll=4)
                def _scale(i):
                    for j in range(D_MODEL // LANES):
                        sl = pl.ds(j * LANES, LANES)
                        rows[b][i, sl] = rows[b][i, sl] * SCALE

                scatter(g, b)

        for b in range(NBUF):
            wait_scatter(b)

    return emb


def kernel(x, weight):
    batch, seq = x.shape
    flat = x.reshape(-1).astype(jnp.int32)
    out = _build(batch * seq)(flat, weight)
    return out.reshape(batch, seq, D_MODEL)


# speedup vs baseline: 1.0744x; 1.0485x over previous
import functools
import math

import jax
import jax.numpy as jnp
from jax import lax
from jax.experimental import pallas as pl
from jax.experimental.pallas import tpu as pltpu
from jax.experimental.pallas import tpu_sc as plsc

D_MODEL = 64
SCALE = math.sqrt(D_MODEL)

NUM_CORES = 2
NUM_SUBCORES = 16
NW = NUM_CORES * NUM_SUBCORES
CHUNK = 160
NBUF = 4
ROW_PAD = 128

TC_BLOCK = 2048


def _pad_scale_body(wt_ref, out_ref):
    out_ref[:, 0:D_MODEL] = wt_ref[...].T * SCALE


@functools.cache
def _build_pad_scale(V: int):
    grid = (V + TC_BLOCK - 1) // TC_BLOCK
    return pl.pallas_call(
        _pad_scale_body,
        grid=(grid,),
        in_specs=[pl.BlockSpec((D_MODEL, TC_BLOCK), lambda i: (0, i))],
        out_specs=pl.BlockSpec((TC_BLOCK, ROW_PAD), lambda i: (i, 0)),
        out_shape=jax.ShapeDtypeStruct((V, ROW_PAD), jnp.float32),
    )


@functools.cache
def _build_gather(B: int, V: int):
    assert B % (NW * CHUNK) == 0
    b_per_w = B // NW
    n_chunks = b_per_w // CHUNK
    assert n_chunks % NBUF == 0 and n_chunks >= 2 * NBUF
    mesh = plsc.VectorSubcoreMesh(
        core_axis_name="c", subcore_axis_name="s",
        num_cores=NUM_CORES, num_subcores=NUM_SUBCORES)

    @functools.partial(
        pl.kernel,
        out_type=jax.ShapeDtypeStruct((B, D_MODEL), jnp.float32),
        mesh=mesh,
        scratch_types=[
            pltpu.VMEM((b_per_w,), jnp.int32),
            [pltpu.VMEM((CHUNK, ROW_PAD), jnp.float32)] * NBUF,
            [pltpu.SemaphoreType.DMA] * NBUF,
            [pltpu.SemaphoreType.DMA] * NBUF,
        ],
        compiler_params=pltpu.CompilerParams(use_tc_tiling_on_sc=False),
    )
    def emb(x_hbm, w_hbm, out_hbm, idx_v, rows, gsem, ssem):
        wid = lax.axis_index("s") * NUM_CORES + lax.axis_index("c")
        base = wid * b_per_w

        def gather(g, b):
            pltpu.async_copy(
                w_hbm.at[idx_v.at[pl.ds(g * CHUNK, CHUNK)]], rows[b], gsem[b])

        def wait_gather(b):
            pltpu.make_async_copy(
                w_hbm.at[idx_v.at[pl.ds(0, CHUNK)]], rows[b], gsem[b]).wait()

        def scatter(g, b):
            pltpu.async_copy(
                rows[b].at[:, pl.ds(0, D_MODEL)],
                out_hbm.at[pl.ds(base + g * CHUNK, CHUNK)], ssem[b])

        def wait_scatter(b):
            pltpu.make_async_copy(
                rows[b].at[:, pl.ds(0, D_MODEL)],
                out_hbm.at[pl.ds(base, CHUNK)], ssem[b]).wait()

        pltpu.sync_copy(x_hbm.at[pl.ds(base, b_per_w)], idx_v)

        for b in range(2):
            gather(b, b)

        @pl.loop(0, n_chunks, step=NBUF)
        def _chunks(g0):
            for b in range(NBUF):
                g = g0 + b
                pf = (b + 2) % NBUF
                @pl.when(g + 2 < n_chunks)
                def _():
                    @pl.when(g >= 2)
                    def _():
                        wait_scatter(pf)
                    gather(g + 2, pf)

                wait_gather(b)
                scatter(g, b)

        for b in range(NBUF):
            wait_scatter(b)

    return emb


def kernel(x, weight):
    batch, seq = x.shape
    vocab, _ = weight.shape
    wt = jnp.swapaxes(weight, 0, 1)
    staged = _build_pad_scale(vocab)(wt)
    flat = x.reshape(-1).astype(jnp.int32)
    out = _build_gather(batch * seq, vocab)(flat, staged)
    return out.reshape(batch, seq, D_MODEL)


# speedup vs baseline: 1.1509x; 1.0712x over previous
import functools
import math

import jax
import jax.numpy as jnp
from jax import lax
from jax.experimental import pallas as pl
from jax.experimental.pallas import tpu as pltpu
from jax.experimental.pallas import tpu_sc as plsc

D_MODEL = 64
SCALE = math.sqrt(D_MODEL)

NUM_CORES = 2
NUM_SUBCORES = 16
NW = NUM_CORES * NUM_SUBCORES
CHUNK = 160
NBUF = 4
ROW_PAD = 128

TC_BLOCK = 2048


def _pad_scale_body(wt_ref, out_ref):
    eye = jax.lax.broadcasted_iota(jnp.int32, (D_MODEL, D_MODEL), 0)
    eye = jnp.where(
        eye == jax.lax.broadcasted_iota(jnp.int32, (D_MODEL, D_MODEL), 1),
        SCALE, 0.0).astype(jnp.float32)
    out_ref[:, 0:D_MODEL] = jax.lax.dot_general(
        wt_ref[...], eye, (((0,), (0,)), ((), ())),
        precision=jax.lax.Precision.HIGHEST,
        preferred_element_type=jnp.float32)


@functools.cache
def _build_pad_scale(V: int):
    grid = (V + TC_BLOCK - 1) // TC_BLOCK
    return pl.pallas_call(
        _pad_scale_body,
        grid=(grid,),
        in_specs=[pl.BlockSpec((D_MODEL, TC_BLOCK), lambda i: (0, i))],
        out_specs=pl.BlockSpec((TC_BLOCK, ROW_PAD), lambda i: (i, 0)),
        out_shape=jax.ShapeDtypeStruct((V, ROW_PAD), jnp.float32),
    )


@functools.cache
def _build_gather(B: int, V: int):
    assert B % (NW * CHUNK) == 0
    b_per_w = B // NW
    n_chunks = b_per_w // CHUNK
    assert n_chunks % NBUF == 0 and n_chunks >= 2 * NBUF
    mesh = plsc.VectorSubcoreMesh(
        core_axis_name="c", subcore_axis_name="s",
        num_cores=NUM_CORES, num_subcores=NUM_SUBCORES)

    @functools.partial(
        pl.kernel,
        out_type=jax.ShapeDtypeStruct((B, ROW_PAD), jnp.float32),
        mesh=mesh,
        scratch_types=[
            pltpu.VMEM((b_per_w,), jnp.int32),
            [pltpu.VMEM((CHUNK, ROW_PAD), jnp.float32)] * NBUF,
            [pltpu.SemaphoreType.DMA] * NBUF,
            [pltpu.SemaphoreType.DMA] * NBUF,
        ],
        compiler_params=pltpu.CompilerParams(use_tc_tiling_on_sc=False),
    )
    def emb(x_hbm, w_hbm, out_hbm, idx_v, rows, gsem, ssem):
        wid = lax.axis_index("s") * NUM_CORES + lax.axis_index("c")
        base = wid * b_per_w

        def gather(g, b):
            pltpu.async_copy(
                w_hbm.at[idx_v.at[pl.ds(g * CHUNK, CHUNK)]], rows[b], gsem[b])

        def wait_gather(b):
            pltpu.make_async_copy(
                w_hbm.at[idx_v.at[pl.ds(0, CHUNK)]], rows[b], gsem[b]).wait()

        def scatter(g, b):
            pltpu.async_copy(
                rows[b], out_hbm.at[pl.ds(base + g * CHUNK, CHUNK)], ssem[b])

        def wait_scatter(b):
            pltpu.make_async_copy(
                rows[b], out_hbm.at[pl.ds(base, CHUNK)], ssem[b]).wait()

        pltpu.sync_copy(x_hbm.at[pl.ds(base, b_per_w)], idx_v)

        for b in range(2):
            gather(b, b)

        @pl.loop(0, n_chunks, step=NBUF)
        def _chunks(g0):
            for b in range(NBUF):
                g = g0 + b
                pf = (b + 2) % NBUF
                @pl.when(g + 2 < n_chunks)
                def _():
                    @pl.when(g >= 2)
                    def _():
                        wait_scatter(pf)
                    gather(g + 2, pf)

                wait_gather(b)
                scatter(g, b)

        for b in range(NBUF):
            wait_scatter(b)

    return emb


def kernel(x, weight):
    batch, seq = x.shape
    vocab, _ = weight.shape
    wt = jnp.swapaxes(weight, 0, 1)
    staged = _build_pad_scale(vocab)(wt)
    flat = x.reshape(-1).astype(jnp.int32)
    out = _build_gather(batch * seq, vocab)(flat, staged)
    return out[:, :D_MODEL].reshape(batch, seq, D_MODEL)
